# Initial kernel scaffold; baseline (speedup 1.0000x reference)
import jax
import jax.numpy as jnp
from jax.experimental import pallas as pl


def kernel(x, edge_index, edge_attr, batch, goal_feat, f1W, f1b, s1W, s1b, g1, be1, f2W, f2b, s2W, s2b, g2, be2, W3, b3, W4, b4, D1W, D1b, D2W, D2b):
    raise NotImplementedError("write your pallas kernel here")


# speedup vs baseline: 1.5611x; 1.5611x over previous
import functools
import math

import jax
import jax.numpy as jnp
from jax import lax
from jax.experimental import pallas as pl
from jax.experimental.pallas import tpu as pltpu

N = 10000
E = 320000
D = 128
DE = 16

_BN = 1.0 / math.sqrt(1.0 + 1e-5)
_RB = 2000
_EB = 4000

_INTERPRET = False


def _nspec(shape, imap):
    return pl.BlockSpec(shape, imap)


def _pre_body(x_ref, wd_ref, ws_ref, td_ref, ts_ref):
    xb = x_ref[...]
    td_ref[...] = jnp.dot(xb, wd_ref[...], preferred_element_type=jnp.float32)
    ts_ref[...] = jnp.dot(xb, ws_ref[...], preferred_element_type=jnp.float32)


def _tc_pre(x, wd, ws):
    return pl.pallas_call(
        _pre_body,
        grid=(N // _RB,),
        in_specs=[
            _nspec((_RB, D), lambda i: (i, 0)),
            _nspec((D, 2 * D), lambda i: (0, 0)),
            _nspec((D, 2 * D), lambda i: (0, 0)),
        ],
        out_specs=[
            _nspec((_RB, 2 * D), lambda i: (i, 0)),
            _nspec((_RB, 2 * D), lambda i: (i, 0)),
        ],
        out_shape=[
            jax.ShapeDtypeStruct((N, 2 * D), jnp.float32),
            jax.ShapeDtypeStruct((N, 2 * D), jnp.float32),
        ],
        interpret=_INTERPRET,
    )(x, wd, ws)


def _mid_body(x_ref, p_ref, g_ref, be_ref, wd_ref, ws_ref,
              h_ref, td_ref, ts_ref):
    agg = (p_ref[0] + p_ref[1]) * _BN
    h = jnp.maximum(x_ref[...] + g_ref[...] * agg + be_ref[...], 0.0)
    h_ref[...] = h
    td_ref[...] = jnp.dot(h, wd_ref[...], preferred_element_type=jnp.float32)
    ts_ref[...] = jnp.dot(h, ws_ref[...], preferred_element_type=jnp.float32)


def _tc_mid(x, parts, g, be, wd, ws):
    return pl.pallas_call(
        _mid_body,
        grid=(N // _RB,),
        in_specs=[
            _nspec((_RB, D), lambda i: (i, 0)),
            _nspec((2, _RB, D), lambda i: (0, i, 0)),
            _nspec((1, D), lambda i: (0, 0)),
            _nspec((1, D), lambda i: (0, 0)),
            _nspec((D, 2 * D), lambda i: (0, 0)),
            _nspec((D, 2 * D), lambda i: (0, 0)),
        ],
        out_specs=[
            _nspec((_RB, D), lambda i: (i, 0)),
            _nspec((_RB, 2 * D), lambda i: (i, 0)),
            _nspec((_RB, 2 * D), lambda i: (i, 0)),
        ],
        out_shape=[
            jax.ShapeDtypeStruct((N, D), jnp.float32),
            jax.ShapeDtypeStruct((N, 2 * D), jnp.float32),
            jax.ShapeDtypeStruct((N, 2 * D), jnp.float32),
        ],
        interpret=_INTERPRET,
    )(x, parts, g.reshape(1, D), be.reshape(1, D), wd, ws)


def _act_body(fs_ref, ea_ref, we_ref, bfs_ref, o_ref):
    pre = fs_ref[...] + jnp.dot(ea_ref[...], we_ref[...],
                                preferred_element_type=jnp.float32)
    pre = pre + bfs_ref[...]
    f = pre[:, :D]
    s = pre[:, D:]
    o_ref[...] = jax.nn.sigmoid(f) * jax.nn.softplus(s)


def _tc_act(fs, ea, we, bfs):
    return pl.pallas_call(
        _act_body,
        grid=(E // _EB,),
        in_specs=[
            _nspec((_EB, 2 * D), lambda i: (i, 0)),
            _nspec((_EB, DE), lambda i: (i, 0)),
            _nspec((DE, 2 * D), lambda i: (0, 0)),
            _nspec((1, 2 * D), lambda i: (0, 0)),
        ],
        out_specs=_nspec((_EB, D), lambda i: (i, 0)),
        out_shape=jax.ShapeDtypeStruct((E, D), jnp.float32),
        interpret=_INTERPRET,
    )(fs, ea, we, bfs.reshape(1, 2 * D))


def _gcn_prep_body(h_ref, p_ref, g_ref, be_ref, degp_ref, w_ref,
                   hv_ref, dinv_ref):
    agg = (p_ref[0] + p_ref[1]) * _BN
    h2 = jnp.maximum(h_ref[...] + g_ref[...] * agg + be_ref[...], 0.0)
    deg = degp_ref[0, :, 0:1] + degp_ref[1, :, 0:1] + 1.0
    dinv = lax.rsqrt(deg)
    hw = jnp.dot(h2, w_ref[...], preferred_element_type=jnp.float32)
    hv_ref[...] = dinv * hw
    dinv_ref[...] = jnp.broadcast_to(dinv, hw.shape)


def _tc_gcn_prep(h, parts, g, be, degp, w):
    return pl.pallas_call(
        _gcn_prep_body,
        grid=(N // _RB,),
        in_specs=[
            _nspec((_RB, D), lambda i: (i, 0)),
            _nspec((2, _RB, D), lambda i: (0, i, 0)),
            _nspec((1, D), lambda i: (0, 0)),
            _nspec((1, D), lambda i: (0, 0)),
            _nspec((2, _RB, DE), lambda i: (0, i, 0)),
            _nspec((D, D), lambda i: (0, 0)),
        ],
        out_specs=[
            _nspec((_RB, D), lambda i: (i, 0)),
            _nspec((_RB, D), lambda i: (i, 0)),
        ],
        out_shape=[
            jax.ShapeDtypeStruct((N, D), jnp.float32),
            jax.ShapeDtypeStruct((N, D), jnp.float32),
        ],
        interpret=_INTERPRET,
    )(h, parts, g.reshape(1, D), be.reshape(1, D), degp, w)


def _gcn_mid_body(p_ref, hv_ref, dinv_ref, b_ref, w_ref, hv4_ref):
    agg = p_ref[0] + p_ref[1] + hv_ref[...]
    h3 = jnp.maximum(dinv_ref[...] * agg + b_ref[...], 0.0)
    hv4_ref[...] = dinv_ref[...] * jnp.dot(
        h3, w_ref[...], preferred_element_type=jnp.float32)


def _tc_gcn_mid(parts, hv, dinv, b, w):
    return pl.pallas_call(
        _gcn_mid_body,
        grid=(N // _RB,),
        in_specs=[
            _nspec((2, _RB, D), lambda i: (0, i, 0)),
            _nspec((_RB, D), lambda i: (i, 0)),
            _nspec((_RB, D), lambda i: (i, 0)),
            _nspec((1, D), lambda i: (0, 0)),
            _nspec((D, D), lambda i: (0, 0)),
        ],
        out_specs=_nspec((_RB, D), lambda i: (i, 0)),
        out_shape=jax.ShapeDtypeStruct((N, D), jnp.float32),
        interpret=_INTERPRET,
    )(parts, hv, dinv, b.reshape(1, D), w)


def _final_body(p_ref, hv_ref, dinv_ref, b4_ref, goal_ref, d1wh_ref,
                d1wg_ref, d1b_ref, d2w_ref, d2b_ref, o_ref):
    agg = p_ref[0] + p_ref[1] + hv_ref[...]
    h4 = jnp.maximum(dinv_ref[...] * agg + b4_ref[...], 0.0)
    gterm = jnp.dot(goal_ref[...], d1wg_ref[...],
                    preferred_element_type=jnp.float32) + d1b_ref[...]
    hid = jnp.maximum(
        jnp.dot(h4, d1wh_ref[...], preferred_element_type=jnp.float32)
        + gterm, 0.0)
    o_ref[...] = jnp.dot(hid, d2w_ref[...],
                         preferred_element_type=jnp.float32) + d2b_ref[...]


def _tc_final(parts, hv, dinv, b4, goal, d1wh, d1wg, d1b, d2w, d2b):
    return pl.pallas_call(
        _final_body,
        grid=(N // _RB,),
        in_specs=[
            _nspec((2, _RB, D), lambda i: (0, i, 0)),
            _nspec((_RB, D), lambda i: (i, 0)),
            _nspec((_RB, D), lambda i: (i, 0)),
            _nspec((1, D), lambda i: (0, 0)),
            _nspec((1, D), lambda i: (0, 0)),
            _nspec((D, D), lambda i: (0, 0)),
            _nspec((D, D), lambda i: (0, 0)),
            _nspec((1, D), lambda i: (0, 0)),
            _nspec((D, 1), lambda i: (0, 0)),
            _nspec((1, 1), lambda i: (0, 0)),
        ],
        out_specs=_nspec((_RB, 1), lambda i: (i, 0)),
        out_shape=jax.ShapeDtypeStruct((N, 1), jnp.float32),
        interpret=_INTERPRET,
    )(parts, hv, dinv, b4.reshape(1, D), goal, d1wh, d1wg,
      d1b.reshape(1, D), d2w, d2b.reshape(1, 1))


def _gather_add(td, ts, dst, src):
    return td[dst] + ts[src]


def _scatter_parts(msg, dst, with_deg):
    seg = jax.ops.segment_sum(msg, dst, num_segments=N)
    parts = jnp.stack([seg, jnp.zeros_like(seg)])
    if not with_deg:
        return parts
    deg = jax.ops.segment_sum(jnp.ones((E, DE), jnp.float32), dst,
                              num_segments=N)
    degp = jnp.stack([deg, jnp.zeros_like(deg)])
    return parts, degp


def _gcn_gather_scatter(hv, src, dst):
    seg = jax.ops.segment_sum(hv[src], dst, num_segments=N)
    return jnp.stack([seg, jnp.zeros_like(seg)])


def kernel(x, edge_index, edge_attr, batch, goal_feat, f1W, f1b, s1W, s1b,
           g1, be1, f2W, f2b, s2W, s2b, g2, be2, W3, b3, W4, b4,
           D1W, D1b, D2W, D2b):
    src = edge_index[0]
    dst = edge_index[1]

    wd1 = jnp.concatenate([f1W[:D], s1W[:D]], axis=1)
    ws1 = jnp.concatenate([f1W[D:2 * D], s1W[D:2 * D]], axis=1)
    we1 = jnp.concatenate([f1W[2 * D:], s1W[2 * D:]], axis=1)
    bfs1 = jnp.concatenate([f1b, s1b])
    wd2 = jnp.concatenate([f2W[:D], s2W[:D]], axis=1)
    ws2 = jnp.concatenate([f2W[D:2 * D], s2W[D:2 * D]], axis=1)
    we2 = jnp.concatenate([f2W[2 * D:], s2W[2 * D:]], axis=1)
    bfs2 = jnp.concatenate([f2b, s2b])

    td1, ts1 = _tc_pre(x, wd1, ws1)
    fs1 = _gather_add(td1, ts1, dst, src)
    msg1 = _tc_act(fs1, edge_attr, we1, bfs1)
    parts1, degp = _scatter_parts(msg1, dst, with_deg=True)

    h1, td2, ts2 = _tc_mid(x, parts1, g1, be1, wd2, ws2)
    fs2 = _gather_add(td2, ts2, dst, src)
    msg2 = _tc_act(fs2, edge_attr, we2, bfs2)
    parts2 = _scatter_parts(msg2, dst, with_deg=False)

    hv3, dinv = _tc_gcn_prep(h1, parts2, g2, be2, degp, W3)
    parts3 = _gcn_gather_scatter(hv3, src, dst)

    hv4 = _tc_gcn_mid(parts3, hv3, dinv, b3, W4)
    parts4 = _gcn_gather_scatter(hv4, src, dst)

    return _tc_final(parts4, hv4, dinv, b4, goal_feat, D1W[:D], D1W[D:],
                     D1b, D2W, D2b)


# speedup vs baseline: 4.8360x; 3.0979x over previous
import functools
import math

import jax
import jax.numpy as jnp
from jax import lax
from jax.experimental import pallas as pl
from jax.experimental.pallas import tpu as pltpu
from jax.experimental.pallas import tpu_sc as plsc

N = 10000
E = 320000
D = 128
DE = 16

_NC = 2
_NS = 16
_NW = _NC * _NS
_EPT = E // _NW
_C = 80
_NCH = _EPT // _C
_RT = 624
_TAIL = N - _RT * _NS

_BN = 1.0 / math.sqrt(1.0 + 1e-5)
_RB = 2000
_EB = 4000

_INTERPRET = False


def _nspec(shape, imap):
    return pl.BlockSpec(shape, imap)


def _pre_body(x_ref, wd_ref, ws_ref, td_ref, ts_ref):
    xb = x_ref[...]
    td_ref[...] = jnp.dot(xb, wd_ref[...], preferred_element_type=jnp.float32)
    ts_ref[...] = jnp.dot(xb, ws_ref[...], preferred_element_type=jnp.float32)


def _tc_pre(x, wd, ws):
    return pl.pallas_call(
        _pre_body,
        grid=(N // _RB,),
        in_specs=[
            _nspec((_RB, D), lambda i: (i, 0)),
            _nspec((D, 2 * D), lambda i: (0, 0)),
            _nspec((D, 2 * D), lambda i: (0, 0)),
        ],
        out_specs=[
            _nspec((_RB, 2 * D), lambda i: (i, 0)),
            _nspec((_RB, 2 * D), lambda i: (i, 0)),
        ],
        out_shape=[
            jax.ShapeDtypeStruct((N, 2 * D), jnp.float32),
            jax.ShapeDtypeStruct((N, 2 * D), jnp.float32),
        ],
        interpret=_INTERPRET,
    )(x, wd, ws)


def _mid_body(x_ref, p_ref, g_ref, be_ref, wd_ref, ws_ref,
              h_ref, td_ref, ts_ref):
    agg = (p_ref[0] + p_ref[1]) * _BN
    h = jnp.maximum(x_ref[...] + g_ref[...] * agg + be_ref[...], 0.0)
    h_ref[...] = h
    td_ref[...] = jnp.dot(h, wd_ref[...], preferred_element_type=jnp.float32)
    ts_ref[...] = jnp.dot(h, ws_ref[...], preferred_element_type=jnp.float32)


def _tc_mid(x, parts, g, be, wd, ws):
    return pl.pallas_call(
        _mid_body,
        grid=(N // _RB,),
        in_specs=[
            _nspec((_RB, D), lambda i: (i, 0)),
            _nspec((2, _RB, D), lambda i: (0, i, 0)),
            _nspec((1, D), lambda i: (0, 0)),
            _nspec((1, D), lambda i: (0, 0)),
            _nspec((D, 2 * D), lambda i: (0, 0)),
            _nspec((D, 2 * D), lambda i: (0, 0)),
        ],
        out_specs=[
            _nspec((_RB, D), lambda i: (i, 0)),
            _nspec((_RB, 2 * D), lambda i: (i, 0)),
            _nspec((_RB, 2 * D), lambda i: (i, 0)),
        ],
        out_shape=[
            jax.ShapeDtypeStruct((N, D), jnp.float32),
            jax.ShapeDtypeStruct((N, 2 * D), jnp.float32),
            jax.ShapeDtypeStruct((N, 2 * D), jnp.float32),
        ],
        interpret=_INTERPRET,
    )(x, parts, g.reshape(1, D), be.reshape(1, D), wd, ws)


def _act_body(fs_ref, ea_ref, we_ref, bfs_ref, o_ref):
    pre = fs_ref[...] + jnp.dot(ea_ref[...], we_ref[...],
                                preferred_element_type=jnp.float32)
    pre = pre + bfs_ref[...]
    f = pre[:, :D]
    s = pre[:, D:]
    o_ref[...] = jax.nn.sigmoid(f) * jax.nn.softplus(s)


def _tc_act(fs, ea, we, bfs):
    return pl.pallas_call(
        _act_body,
        grid=(E // _EB,),
        in_specs=[
            _nspec((_EB, 2 * D), lambda i: (i, 0)),
            _nspec((_EB, DE), lambda i: (i, 0)),
            _nspec((DE, 2 * D), lambda i: (0, 0)),
            _nspec((1, 2 * D), lambda i: (0, 0)),
        ],
        out_specs=_nspec((_EB, D), lambda i: (i, 0)),
        out_shape=jax.ShapeDtypeStruct((E, D), jnp.float32),
        interpret=_INTERPRET,
    )(fs, ea, we, bfs.reshape(1, 2 * D))


def _gcn_prep_body(h_ref, p_ref, g_ref, be_ref, deg_ref, w_ref,
                   hv_ref, dinv_ref):
    agg = (p_ref[0] + p_ref[1]) * _BN
    h2 = jnp.maximum(h_ref[...] + g_ref[...] * agg + be_ref[...], 0.0)
    deg = deg_ref[...] + 1.0
    dinv = lax.rsqrt(deg)
    hw = jnp.dot(h2, w_ref[...], preferred_element_type=jnp.float32)
    hv_ref[...] = dinv * hw
    dinv_ref[...] = jnp.broadcast_to(dinv, hw.shape)


def _tc_gcn_prep(h, parts, g, be, deg, w):
    return pl.pallas_call(
        _gcn_prep_body,
        grid=(N // _RB,),
        in_specs=[
            _nspec((_RB, D), lambda i: (i, 0)),
            _nspec((2, _RB, D), lambda i: (0, i, 0)),
            _nspec((1, D), lambda i: (0, 0)),
            _nspec((1, D), lambda i: (0, 0)),
            _nspec((_RB, 1), lambda i: (i, 0)),
            _nspec((D, D), lambda i: (0, 0)),
        ],
        out_specs=[
            _nspec((_RB, D), lambda i: (i, 0)),
            _nspec((_RB, D), lambda i: (i, 0)),
        ],
        out_shape=[
            jax.ShapeDtypeStruct((N, D), jnp.float32),
            jax.ShapeDtypeStruct((N, D), jnp.float32),
        ],
        interpret=_INTERPRET,
    )(h, parts, g.reshape(1, D), be.reshape(1, D), deg, w)


def _gcn_mid_body(p_ref, hv_ref, dinv_ref, b_ref, w_ref, hv4_ref):
    agg = p_ref[0] + p_ref[1] + hv_ref[...]
    h3 = jnp.maximum(dinv_ref[...] * agg + b_ref[...], 0.0)
    hv4_ref[...] = dinv_ref[...] * jnp.dot(
        h3, w_ref[...], preferred_element_type=jnp.float32)


def _tc_gcn_mid(parts, hv, dinv, b, w):
    return pl.pallas_call(
        _gcn_mid_body,
        grid=(N // _RB,),
        in_specs=[
            _nspec((2, _RB, D), lambda i: (0, i, 0)),
            _nspec((_RB, D), lambda i: (i, 0)),
            _nspec((_RB, D), lambda i: (i, 0)),
            _nspec((1, D), lambda i: (0, 0)),
            _nspec((D, D), lambda i: (0, 0)),
        ],
        out_specs=_nspec((_RB, D), lambda i: (i, 0)),
        out_shape=jax.ShapeDtypeStruct((N, D), jnp.float32),
        interpret=_INTERPRET,
    )(parts, hv, dinv, b.reshape(1, D), w)


def _final_body(p_ref, hv_ref, dinv_ref, b4_ref, goal_ref, d1wh_ref,
                d1wg_ref, d1b_ref, d2w_ref, d2b_ref, o_ref):
    agg = p_ref[0] + p_ref[1] + hv_ref[...]
    h4 = jnp.maximum(dinv_ref[...] * agg + b4_ref[...], 0.0)
    gterm = jnp.dot(goal_ref[...], d1wg_ref[...],
                    preferred_element_type=jnp.float32) + d1b_ref[...]
    hid = jnp.maximum(
        jnp.dot(h4, d1wh_ref[...], preferred_element_type=jnp.float32)
        + gterm, 0.0)
    o_ref[...] = jnp.dot(hid, d2w_ref[...],
                         preferred_element_type=jnp.float32) + d2b_ref[...]


def _tc_final(parts, hv, dinv, b4, goal, d1wh, d1wg, d1b, d2w, d2b):
    return pl.pallas_call(
        _final_body,
        grid=(N // _RB,),
        in_specs=[
            _nspec((2, _RB, D), lambda i: (0, i, 0)),
            _nspec((_RB, D), lambda i: (i, 0)),
            _nspec((_RB, D), lambda i: (i, 0)),
            _nspec((1, D), lambda i: (0, 0)),
            _nspec((1, D), lambda i: (0, 0)),
            _nspec((D, D), lambda i: (0, 0)),
            _nspec((D, D), lambda i: (0, 0)),
            _nspec((1, D), lambda i: (0, 0)),
            _nspec((D, 1), lambda i: (0, 0)),
            _nspec((1, 1), lambda i: (0, 0)),
        ],
        out_specs=_nspec((_RB, 1), lambda i: (i, 0)),
        out_shape=jax.ShapeDtypeStruct((N, 1), jnp.float32),
        interpret=_INTERPRET,
    )(parts, hv, dinv, b4.reshape(1, D), goal, d1wh, d1wg,
      d1b.reshape(1, D), d2w, d2b.reshape(1, 1))


_SC_MESH = plsc.VectorSubcoreMesh(core_axis_name="c", subcore_axis_name="s",
                                  num_cores=_NC, num_subcores=_NS)


def _rowcopy(sid, src_ref, dst_ref):
    pltpu.sync_copy(src_ref.at[pl.ds(sid * _RT, _RT)],
                    dst_ref.at[pl.ds(sid * _RT, _RT)])

    @pl.when(sid == _NS - 1)
    def _():
        pltpu.sync_copy(src_ref.at[pl.ds(_RT * _NS, _TAIL)],
                        dst_ref.at[pl.ds(_RT * _NS, _TAIL)])


def _scat_body(msg_hbm, dstx_hbm, zero_hbm, out_hbm, didx, buf, accum):
    cid = lax.axis_index("c")
    sid = lax.axis_index("s")
    base = (cid * _NS + sid) * _EPT
    pltpu.sync_copy(dstx_hbm.at[cid, sid], didx)
    _rowcopy(sid, zero_hbm, accum)
    plsc.subcore_barrier()

    def step(j, carry):
        pltpu.sync_copy(msg_hbm.at[pl.ds(base + j * _C, _C)], buf)
        pltpu.sync_copy(buf, accum.at[didx.at[j]], add=True)
        return carry

    lax.fori_loop(0, _NCH, step, 0)
    plsc.subcore_barrier()
    _rowcopy(sid, accum, out_hbm.at[cid])


_sc_scat = pl.kernel(
    _scat_body,
    out_type=jax.ShapeDtypeStruct((_NC, N, D), jnp.float32),
    mesh=_SC_MESH,
    scratch_types=[
        pltpu.VMEM((_NCH, _C), jnp.int32),
        pltpu.VMEM((_C, D), jnp.float32),
        pltpu.VMEM_SHARED((N, D), jnp.float32),
    ],
)


def _gcn_body(hv_hbm, srcx_hbm, dstx_hbm, zero_hbm, out_hbm,
              sidx, didx, buf, accum):
    cid = lax.axis_index("c")
    sid = lax.axis_index("s")
    pltpu.sync_copy(srcx_hbm.at[cid, sid], sidx)
    pltpu.sync_copy(dstx_hbm.at[cid, sid], didx)
    _rowcopy(sid, zero_hbm, accum)
    plsc.subcore_barrier()

    def step(j, carry):
        pltpu.sync_copy(hv_hbm.at[sidx.at[j]], buf)
        pltpu.sync_copy(buf, accum.at[didx.at[j]], add=True)
        return carry

    lax.fori_loop(0, _NCH, step, 0)
    plsc.subcore_barrier()
    _rowcopy(sid, accum, out_hbm.at[cid])


_sc_gcn = pl.kernel(
    _gcn_body,
    out_type=jax.ShapeDtypeStruct((_NC, N, D), jnp.float32),
    mesh=_SC_MESH,
    scratch_types=[
        pltpu.VMEM((_NCH, _C), jnp.int32),
        pltpu.VMEM((_NCH, _C), jnp.int32),
        pltpu.VMEM((_C, D), jnp.float32),
        pltpu.VMEM_SHARED((N, D), jnp.float32),
    ],
)


def _ga_body(td_hbm, ts_hbm, dstx_hbm, srcx_hbm, out_hbm,
             didx, sidx, bufa, bufb):
    cid = lax.axis_index("c")
    sid = lax.axis_index("s")
    base = (cid * _NS + sid) * _EPT
    pltpu.sync_copy(dstx_hbm.at[cid, sid], didx)
    pltpu.sync_copy(srcx_hbm.at[cid, sid], sidx)

    def step(j, carry):
        pltpu.sync_copy(td_hbm.at[didx.at[j]], bufa)
        pltpu.sync_copy(ts_hbm.at[sidx.at[j]], bufb)

        def row(r, c2):
            for k in range(2 * D // 16):
                sl = pl.ds(k * 16, 16)
                bufa[r, sl] = bufa[r, sl] + bufb[r, sl]
            return c2

        lax.fori_loop(0, _C, row, 0)
        pltpu.sync_copy(bufa, out_hbm.at[pl.ds(base + j * _C, _C)])
        return carry

    lax.fori_loop(0, _NCH, step, 0)


_sc_gather_add = pl.kernel(
    _ga_body,
    out_type=jax.ShapeDtypeStruct((E, 2 * D), jnp.float32),
    mesh=_SC_MESH,
    scratch_types=[
        pltpu.VMEM((_NCH, _C), jnp.int32),
        pltpu.VMEM((_NCH, _C), jnp.int32),
        pltpu.VMEM((_C, 2 * D), jnp.float32),
        pltpu.VMEM((_C, 2 * D), jnp.float32),
    ],
)


def _gather_add(td, ts, dst, src):
    return td[dst] + ts[src]


def kernel(x, edge_index, edge_attr, batch, goal_feat, f1W, f1b, s1W, s1b,
           g1, be1, f2W, f2b, s2W, s2b, g2, be2, W3, b3, W4, b4,
           D1W, D1b, D2W, D2b):
    src = edge_index[0]
    dst = edge_index[1]
    srcx = src.reshape(_NC, _NS, _NCH, _C)
    dstx = dst.reshape(_NC, _NS, _NCH, _C)
    zero_nd = jnp.zeros((N, D), jnp.float32)
    deg = jax.ops.segment_sum(jnp.ones((E,), jnp.float32), dst,
                              num_segments=N).reshape(N, 1)

    wd1 = jnp.concatenate([f1W[:D], s1W[:D]], axis=1)
    ws1 = jnp.concatenate([f1W[D:2 * D], s1W[D:2 * D]], axis=1)
    we1 = jnp.concatenate([f1W[2 * D:], s1W[2 * D:]], axis=1)
    bfs1 = jnp.concatenate([f1b, s1b])
    wd2 = jnp.concatenate([f2W[:D], s2W[:D]], axis=1)
    ws2 = jnp.concatenate([f2W[D:2 * D], s2W[D:2 * D]], axis=1)
    we2 = jnp.concatenate([f2W[2 * D:], s2W[2 * D:]], axis=1)
    bfs2 = jnp.concatenate([f2b, s2b])

    td1, ts1 = _tc_pre(x, wd1, ws1)
    fs1 = _sc_gather_add(td1, ts1, dstx, srcx)
    msg1 = _tc_act(fs1, edge_attr, we1, bfs1)
    parts1 = _sc_scat(msg1, dstx, zero_nd)

    h1, td2, ts2 = _tc_mid(x, parts1, g1, be1, wd2, ws2)
    fs2 = _sc_gather_add(td2, ts2, dstx, srcx)
    msg2 = _tc_act(fs2, edge_attr, we2, bfs2)
    parts2 = _sc_scat(msg2, dstx, zero_nd)

    hv3, dinv = _tc_gcn_prep(h1, parts2, g2, be2, deg, W3)
    parts3 = _sc_gcn(hv3, srcx, dstx, zero_nd)

    hv4 = _tc_gcn_mid(parts3, hv3, dinv, b3, W4)
    parts4 = _sc_gcn(hv4, srcx, dstx, zero_nd)

    return _tc_final(parts4, hv4, dinv, b4, goal_feat, D1W[:D], D1W[D:],
                     D1b, D2W, D2b)


# speedup vs baseline: 5.8645x; 1.2127x over previous
import functools
import math

import jax
import jax.numpy as jnp
from jax import lax
from jax.experimental import pallas as pl
from jax.experimental.pallas import tpu as pltpu
from jax.experimental.pallas import tpu_sc as plsc

N = 10000
E = 320000
D = 128
DE = 16

_NC = 2
_NS = 16
_NW = _NC * _NS
_EPT = E // _NW
_C = 80
_NCH = _EPT // _C
_RT = 624
_TAIL = N - _RT * _NS

_BN = 1.0 / math.sqrt(1.0 + 1e-5)
_RB = 2000
_EB = 4000

_INTERPRET = False


def _nspec(shape, imap):
    return pl.BlockSpec(shape, imap)


def _pack2(t):
    f = lax.bitcast_convert_type(t[:, :D], jnp.uint32)
    s = lax.bitcast_convert_type(t[:, D:], jnp.uint32)
    f = (f + jnp.uint32(0x7FFF) + ((f >> 16) & jnp.uint32(1))) >> 16
    s = (s + jnp.uint32(0x7FFF) + ((s >> 16) & jnp.uint32(1))) & jnp.uint32(
        0xFFFF0000)
    return f | s


def _unpack_f(w):
    return lax.bitcast_convert_type(w << 16, jnp.float32)


def _unpack_s(w):
    return lax.bitcast_convert_type(w & jnp.uint32(0xFFFF0000), jnp.float32)


def _pre_body(x_ref, wd_ref, ws_ref, td_ref, ts_ref):
    xb = x_ref[...]
    td_ref[...] = _pack2(
        jnp.dot(xb, wd_ref[...], preferred_element_type=jnp.float32))
    ts_ref[...] = _pack2(
        jnp.dot(xb, ws_ref[...], preferred_element_type=jnp.float32))


def _tc_pre(x, wd, ws):
    return pl.pallas_call(
        _pre_body,
        grid=(N // _RB,),
        in_specs=[
            _nspec((_RB, D), lambda i: (i, 0)),
            _nspec((D, 2 * D), lambda i: (0, 0)),
            _nspec((D, 2 * D), lambda i: (0, 0)),
        ],
        out_specs=[
            _nspec((_RB, D), lambda i: (i, 0)),
            _nspec((_RB, D), lambda i: (i, 0)),
        ],
        out_shape=[
            jax.ShapeDtypeStruct((N, D), jnp.uint32),
            jax.ShapeDtypeStruct((N, D), jnp.uint32),
        ],
        interpret=_INTERPRET,
    )(x, wd, ws)


def _mid_body(x_ref, p_ref, g_ref, be_ref, wd_ref, ws_ref,
              h_ref, td_ref, ts_ref):
    agg = (p_ref[0] + p_ref[1]) * _BN
    h = jnp.maximum(x_ref[...] + g_ref[...] * agg + be_ref[...], 0.0)
    h_ref[...] = h
    td_ref[...] = _pack2(
        jnp.dot(h, wd_ref[...], preferred_element_type=jnp.float32))
    ts_ref[...] = _pack2(
        jnp.dot(h, ws_ref[...], preferred_element_type=jnp.float32))


def _tc_mid(x, parts, g, be, wd, ws):
    return pl.pallas_call(
        _mid_body,
        grid=(N // _RB,),
        in_specs=[
            _nspec((_RB, D), lambda i: (i, 0)),
            _nspec((2, _RB, D), lambda i: (0, i, 0)),
            _nspec((1, D), lambda i: (0, 0)),
            _nspec((1, D), lambda i: (0, 0)),
            _nspec((D, 2 * D), lambda i: (0, 0)),
            _nspec((D, 2 * D), lambda i: (0, 0)),
        ],
        out_specs=[
            _nspec((_RB, D), lambda i: (i, 0)),
            _nspec((_RB, D), lambda i: (i, 0)),
            _nspec((_RB, D), lambda i: (i, 0)),
        ],
        out_shape=[
            jax.ShapeDtypeStruct((N, D), jnp.float32),
            jax.ShapeDtypeStruct((N, D), jnp.uint32),
            jax.ShapeDtypeStruct((N, D), jnp.uint32),
        ],
        interpret=_INTERPRET,
    )(x, parts, g.reshape(1, D), be.reshape(1, D), wd, ws)


def _act_body(fd_ref, fs_ref, ea_ref, we_ref, bfs_ref, o_ref):
    wd_ = fd_ref[...]
    ws_ = fs_ref[...]
    g = jnp.dot(ea_ref[...], we_ref[...],
                preferred_element_type=jnp.float32) + bfs_ref[...]
    f = _unpack_f(wd_) + _unpack_f(ws_) + g[:, :D]
    s = _unpack_s(wd_) + _unpack_s(ws_) + g[:, D:]
    o_ref[...] = jax.nn.sigmoid(f) * jax.nn.softplus(s)


def _tc_act(fd, fs, ea, we, bfs):
    return pl.pallas_call(
        _act_body,
        grid=(E // _EB,),
        in_specs=[
            _nspec((_EB, D), lambda i: (i, 0)),
            _nspec((_EB, D), lambda i: (i, 0)),
            _nspec((_EB, DE), lambda i: (i, 0)),
            _nspec((DE, 2 * D), lambda i: (0, 0)),
            _nspec((1, 2 * D), lambda i: (0, 0)),
        ],
        out_specs=_nspec((_EB, D), lambda i: (i, 0)),
        out_shape=jax.ShapeDtypeStruct((E, D), jnp.float32),
        interpret=_INTERPRET,
    )(fd, fs, ea, we, bfs.reshape(1, 2 * D))


def _gcn_prep_body(h_ref, p_ref, g_ref, be_ref, deg_ref, w_ref,
                   hv_ref, dinv_ref):
    agg = (p_ref[0] + p_ref[1]) * _BN
    h2 = jnp.maximum(h_ref[...] + g_ref[...] * agg + be_ref[...], 0.0)
    deg = deg_ref[...] + 1.0
    dinv = lax.rsqrt(deg)
    hw = jnp.dot(h2, w_ref[...], preferred_element_type=jnp.float32)
    hv_ref[...] = dinv * hw
    dinv_ref[...] = jnp.broadcast_to(dinv, hw.shape)


def _tc_gcn_prep(h, parts, g, be, deg, w):
    return pl.pallas_call(
        _gcn_prep_body,
        grid=(N // _RB,),
        in_specs=[
            _nspec((_RB, D), lambda i: (i, 0)),
            _nspec((2, _RB, D), lambda i: (0, i, 0)),
            _nspec((1, D), lambda i: (0, 0)),
            _nspec((1, D), lambda i: (0, 0)),
            _nspec((_RB, 1), lambda i: (i, 0)),
            _nspec((D, D), lambda i: (0, 0)),
        ],
        out_specs=[
            _nspec((_RB, D), lambda i: (i, 0)),
            _nspec((_RB, D), lambda i: (i, 0)),
        ],
        out_shape=[
            jax.ShapeDtypeStruct((N, D), jnp.float32),
            jax.ShapeDtypeStruct((N, D), jnp.float32),
        ],
        interpret=_INTERPRET,
    )(h, parts, g.reshape(1, D), be.reshape(1, D), deg, w)


def _gcn_mid_body(p_ref, hv_ref, dinv_ref, b_ref, w_ref, hv4_ref):
    agg = p_ref[0] + p_ref[1] + hv_ref[...]
    h3 = jnp.maximum(dinv_ref[...] * agg + b_ref[...], 0.0)
    hv4_ref[...] = dinv_ref[...] * jnp.dot(
        h3, w_ref[...], preferred_element_type=jnp.float32)


def _tc_gcn_mid(parts, hv, dinv, b, w):
    return pl.pallas_call(
        _gcn_mid_body,
        grid=(N // _RB,),
        in_specs=[
            _nspec((2, _RB, D), lambda i: (0, i, 0)),
            _nspec((_RB, D), lambda i: (i, 0)),
            _nspec((_RB, D), lambda i: (i, 0)),
            _nspec((1, D), lambda i: (0, 0)),
            _nspec((D, D), lambda i: (0, 0)),
        ],
        out_specs=_nspec((_RB, D), lambda i: (i, 0)),
        out_shape=jax.ShapeDtypeStruct((N, D), jnp.float32),
        interpret=_INTERPRET,
    )(parts, hv, dinv, b.reshape(1, D), w)


def _final_body(p_ref, hv_ref, dinv_ref, b4_ref, goal_ref, d1wh_ref,
                d1wg_ref, d1b_ref, d2w_ref, d2b_ref, o_ref):
    agg = p_ref[0] + p_ref[1] + hv_ref[...]
    h4 = jnp.maximum(dinv_ref[...] * agg + b4_ref[...], 0.0)
    gterm = jnp.dot(goal_ref[...], d1wg_ref[...],
                    preferred_element_type=jnp.float32) + d1b_ref[...]
    hid = jnp.maximum(
        jnp.dot(h4, d1wh_ref[...], preferred_element_type=jnp.float32)
        + gterm, 0.0)
    o_ref[...] = jnp.dot(hid, d2w_ref[...],
                         preferred_element_type=jnp.float32) + d2b_ref[...]


def _tc_final(parts, hv, dinv, b4, goal, d1wh, d1wg, d1b, d2w, d2b):
    return pl.pallas_call(
        _final_body,
        grid=(N // _RB,),
        in_specs=[
            _nspec((2, _RB, D), lambda i: (0, i, 0)),
            _nspec((_RB, D), lambda i: (i, 0)),
            _nspec((_RB, D), lambda i: (i, 0)),
            _nspec((1, D), lambda i: (0, 0)),
            _nspec((1, D), lambda i: (0, 0)),
            _nspec((D, D), lambda i: (0, 0)),
            _nspec((D, D), lambda i: (0, 0)),
            _nspec((1, D), lambda i: (0, 0)),
            _nspec((D, 1), lambda i: (0, 0)),
            _nspec((1, 1), lambda i: (0, 0)),
        ],
        out_specs=_nspec((_RB, 1), lambda i: (i, 0)),
        out_shape=jax.ShapeDtypeStruct((N, 1), jnp.float32),
        interpret=_INTERPRET,
    )(parts, hv, dinv, b4.reshape(1, D), goal, d1wh, d1wg,
      d1b.reshape(1, D), d2w, d2b.reshape(1, 1))


_SC_MESH = plsc.VectorSubcoreMesh(core_axis_name="c", subcore_axis_name="s",
                                  num_cores=_NC, num_subcores=_NS)


def _rowcopy(sid, src_ref, dst_ref):
    pltpu.sync_copy(src_ref.at[pl.ds(sid * _RT, _RT)],
                    dst_ref.at[pl.ds(sid * _RT, _RT)])

    @pl.when(sid == _NS - 1)
    def _():
        pltpu.sync_copy(src_ref.at[pl.ds(_RT * _NS, _TAIL)],
                        dst_ref.at[pl.ds(_RT * _NS, _TAIL)])


def _scat_body(msg_hbm, dstx_hbm, zero_hbm, out_hbm, didx, buf, accum):
    cid = lax.axis_index("c")
    sid = lax.axis_index("s")
    base = (cid * _NS + sid) * _EPT
    pltpu.sync_copy(dstx_hbm.at[cid, sid], didx)
    _rowcopy(sid, zero_hbm, accum)
    plsc.subcore_barrier()

    def step(j, carry):
        pltpu.sync_copy(msg_hbm.at[pl.ds(base + j * _C, _C)], buf)
        pltpu.sync_copy(buf, accum.at[didx.at[j]], add=True)
        return carry

    lax.fori_loop(0, _NCH, step, 0)
    plsc.subcore_barrier()
    _rowcopy(sid, accum, out_hbm.at[cid])


_sc_scat = pl.kernel(
    _scat_body,
    out_type=jax.ShapeDtypeStruct((_NC, N, D), jnp.float32),
    mesh=_SC_MESH,
    scratch_types=[
        pltpu.VMEM((_NCH, _C), jnp.int32),
        pltpu.VMEM((_C, D), jnp.float32),
        pltpu.VMEM_SHARED((N, D), jnp.float32),
    ],
)


def _gcn_body(hv_hbm, srcx_hbm, dstx_hbm, zero_hbm, out_hbm,
              sidx, didx, buf, accum):
    cid = lax.axis_index("c")
    sid = lax.axis_index("s")
    pltpu.sync_copy(srcx_hbm.at[cid, sid], sidx)
    pltpu.sync_copy(dstx_hbm.at[cid, sid], didx)
    _rowcopy(sid, zero_hbm, accum)
    plsc.subcore_barrier()

    def step(j, carry):
        pltpu.sync_copy(hv_hbm.at[sidx.at[j]], buf)
        pltpu.sync_copy(buf, accum.at[didx.at[j]], add=True)
        return carry

    lax.fori_loop(0, _NCH, step, 0)
    plsc.subcore_barrier()
    _rowcopy(sid, accum, out_hbm.at[cid])


_sc_gcn = pl.kernel(
    _gcn_body,
    out_type=jax.ShapeDtypeStruct((_NC, N, D), jnp.float32),
    mesh=_SC_MESH,
    scratch_types=[
        pltpu.VMEM((_NCH, _C), jnp.int32),
        pltpu.VMEM((_NCH, _C), jnp.int32),
        pltpu.VMEM((_C, D), jnp.float32),
        pltpu.VMEM_SHARED((N, D), jnp.float32),
    ],
)


def _ga_body(td_hbm, ts_hbm, dstx_hbm, srcx_hbm, outd_hbm, outs_hbm,
             didx, sidx, bufa, bufb):
    cid = lax.axis_index("c")
    sid = lax.axis_index("s")
    base = (cid * _NS + sid) * _EPT
    pltpu.sync_copy(dstx_hbm.at[cid, sid], didx)
    pltpu.sync_copy(srcx_hbm.at[cid, sid], sidx)

    def step(j, carry):
        rows = pl.ds(base + j * _C, _C)
        pltpu.sync_copy(td_hbm.at[didx.at[j]], bufa)
        pltpu.sync_copy(ts_hbm.at[sidx.at[j]], bufb)
        pltpu.sync_copy(bufa, outd_hbm.at[rows])
        pltpu.sync_copy(bufb, outs_hbm.at[rows])
        return carry

    lax.fori_loop(0, _NCH, step, 0)


_sc_gather_add = pl.kernel(
    _ga_body,
    out_type=[jax.ShapeDtypeStruct((E, D), jnp.uint32),
              jax.ShapeDtypeStruct((E, D), jnp.uint32)],
    mesh=_SC_MESH,
    scratch_types=[
        pltpu.VMEM((_NCH, _C), jnp.int32),
        pltpu.VMEM((_NCH, _C), jnp.int32),
        pltpu.VMEM((_C, D), jnp.uint32),
        pltpu.VMEM((_C, D), jnp.uint32),
    ],
)


def _gather_add(td, ts, dst, src):
    return td[dst] + ts[src]


def kernel(x, edge_index, edge_attr, batch, goal_feat, f1W, f1b, s1W, s1b,
           g1, be1, f2W, f2b, s2W, s2b, g2, be2, W3, b3, W4, b4,
           D1W, D1b, D2W, D2b):
    src = edge_index[0]
    dst = edge_index[1]
    srcx = src.reshape(_NC, _NS, _NCH, _C)
    dstx = dst.reshape(_NC, _NS, _NCH, _C)
    zero_nd = jnp.zeros((N, D), jnp.float32)
    deg = jax.ops.segment_sum(jnp.ones((E,), jnp.float32), dst,
                              num_segments=N).reshape(N, 1)

    wd1 = jnp.concatenate([f1W[:D], s1W[:D]], axis=1)
    ws1 = jnp.concatenate([f1W[D:2 * D], s1W[D:2 * D]], axis=1)
    we1 = jnp.concatenate([f1W[2 * D:], s1W[2 * D:]], axis=1)
    bfs1 = jnp.concatenate([f1b, s1b])
    wd2 = jnp.concatenate([f2W[:D], s2W[:D]], axis=1)
    ws2 = jnp.concatenate([f2W[D:2 * D], s2W[D:2 * D]], axis=1)
    we2 = jnp.concatenate([f2W[2 * D:], s2W[2 * D:]], axis=1)
    bfs2 = jnp.concatenate([f2b, s2b])

    td1, ts1 = _tc_pre(x, wd1, ws1)
    fd1, fs1 = _sc_gather_add(td1, ts1, dstx, srcx)
    msg1 = _tc_act(fd1, fs1, edge_attr, we1, bfs1)
    parts1 = _sc_scat(msg1, dstx, zero_nd)

    h1, td2, ts2 = _tc_mid(x, parts1, g1, be1, wd2, ws2)
    fd2, fs2 = _sc_gather_add(td2, ts2, dstx, srcx)
    msg2 = _tc_act(fd2, fs2, edge_attr, we2, bfs2)
    parts2 = _sc_scat(msg2, dstx, zero_nd)

    hv3, dinv = _tc_gcn_prep(h1, parts2, g2, be2, deg, W3)
    parts3 = _sc_gcn(hv3, srcx, dstx, zero_nd)

    hv4 = _tc_gcn_mid(parts3, hv3, dinv, b3, W4)
    parts4 = _sc_gcn(hv4, srcx, dstx, zero_nd)

    return _tc_final(parts4, hv4, dinv, b4, goal_feat, D1W[:D], D1W[D:],
                     D1b, D2W, D2b)


# speedup vs baseline: 6.5281x; 1.1132x over previous
import functools
import math

import jax
import jax.numpy as jnp
from jax import lax
from jax.experimental import pallas as pl
from jax.experimental.pallas import tpu as pltpu
from jax.experimental.pallas import tpu_sc as plsc

N = 10000
E = 320000
D = 128
DE = 16

_NC = 2
_NS = 16
_NW = _NC * _NS
_EPT = E // _NW
_C = 80
_NCH = _EPT // _C
_RT = 624
_TAIL = N - _RT * _NS

_BN = 1.0 / math.sqrt(1.0 + 1e-5)
_RB = 2000
_EB = 4000

_INTERPRET = False


def _nspec(shape, imap):
    return pl.BlockSpec(shape, imap)


def _pack2(t):
    f = lax.bitcast_convert_type(t[:, :D], jnp.uint32)
    s = lax.bitcast_convert_type(t[:, D:], jnp.uint32)
    f = (f + jnp.uint32(0x7FFF) + ((f >> 16) & jnp.uint32(1))) >> 16
    s = (s + jnp.uint32(0x7FFF) + ((s >> 16) & jnp.uint32(1))) & jnp.uint32(
        0xFFFF0000)
    return f | s


def _unpack_f(w):
    return lax.bitcast_convert_type(w << 16, jnp.float32)


def _unpack_s(w):
    return lax.bitcast_convert_type(w & jnp.uint32(0xFFFF0000), jnp.float32)


def _pre_body(x_ref, wd_ref, ws_ref, td_ref, ts_ref):
    xb = x_ref[...]
    td_ref[...] = _pack2(
        jnp.dot(xb, wd_ref[...], preferred_element_type=jnp.float32))
    ts_ref[...] = _pack2(
        jnp.dot(xb, ws_ref[...], preferred_element_type=jnp.float32))


def _tc_pre(x, wd, ws):
    return pl.pallas_call(
        _pre_body,
        grid=(N // _RB,),
        in_specs=[
            _nspec((_RB, D), lambda i: (i, 0)),
            _nspec((D, 2 * D), lambda i: (0, 0)),
            _nspec((D, 2 * D), lambda i: (0, 0)),
        ],
        out_specs=[
            _nspec((_RB, D), lambda i: (i, 0)),
            _nspec((_RB, D), lambda i: (i, 0)),
        ],
        out_shape=[
            jax.ShapeDtypeStruct((N, D), jnp.uint32),
            jax.ShapeDtypeStruct((N, D), jnp.uint32),
        ],
        interpret=_INTERPRET,
    )(x, wd, ws)


def _mid_body(x_ref, p_ref, g_ref, be_ref, wd_ref, ws_ref,
              h_ref, td_ref, ts_ref):
    agg = (p_ref[0] + p_ref[1]) * _BN
    h = jnp.maximum(x_ref[...] + g_ref[...] * agg + be_ref[...], 0.0)
    h_ref[...] = h
    td_ref[...] = _pack2(
        jnp.dot(h, wd_ref[...], preferred_element_type=jnp.float32))
    ts_ref[...] = _pack2(
        jnp.dot(h, ws_ref[...], preferred_element_type=jnp.float32))


def _tc_mid(x, parts, g, be, wd, ws):
    return pl.pallas_call(
        _mid_body,
        grid=(N // _RB,),
        in_specs=[
            _nspec((_RB, D), lambda i: (i, 0)),
            _nspec((2, _RB, D), lambda i: (0, i, 0)),
            _nspec((1, D), lambda i: (0, 0)),
            _nspec((1, D), lambda i: (0, 0)),
            _nspec((D, 2 * D), lambda i: (0, 0)),
            _nspec((D, 2 * D), lambda i: (0, 0)),
        ],
        out_specs=[
            _nspec((_RB, D), lambda i: (i, 0)),
            _nspec((_RB, D), lambda i: (i, 0)),
            _nspec((_RB, D), lambda i: (i, 0)),
        ],
        out_shape=[
            jax.ShapeDtypeStruct((N, D), jnp.float32),
            jax.ShapeDtypeStruct((N, D), jnp.uint32),
            jax.ShapeDtypeStruct((N, D), jnp.uint32),
        ],
        interpret=_INTERPRET,
    )(x, parts, g.reshape(1, D), be.reshape(1, D), wd, ws)


def _act_body(fd_ref, fs_ref, ea_ref, we_ref, bfs_ref, o_ref):
    wd_ = fd_ref[...]
    ws_ = fs_ref[...]
    g = jnp.dot(ea_ref[...], we_ref[...],
                preferred_element_type=jnp.float32) + bfs_ref[...]
    f = _unpack_f(wd_) + _unpack_f(ws_) + g[:, :D]
    s = _unpack_s(wd_) + _unpack_s(ws_) + g[:, D:]
    o_ref[...] = jax.nn.sigmoid(f) * jax.nn.softplus(s)


def _tc_act(fd, fs, ea, we, bfs):
    return pl.pallas_call(
        _act_body,
        grid=(E // _EB,),
        in_specs=[
            _nspec((_EB, D), lambda i: (i, 0)),
            _nspec((_EB, D), lambda i: (i, 0)),
            _nspec((_EB, DE), lambda i: (i, 0)),
            _nspec((DE, 2 * D), lambda i: (0, 0)),
            _nspec((1, 2 * D), lambda i: (0, 0)),
        ],
        out_specs=_nspec((_EB, D), lambda i: (i, 0)),
        out_shape=jax.ShapeDtypeStruct((E, D), jnp.float32),
        interpret=_INTERPRET,
    )(fd, fs, ea, we, bfs.reshape(1, 2 * D))


def _gcn_prep_body(h_ref, p_ref, g_ref, be_ref, deg_ref, w_ref,
                   hv_ref, dinv_ref):
    agg = (p_ref[0] + p_ref[1]) * _BN
    h2 = jnp.maximum(h_ref[...] + g_ref[...] * agg + be_ref[...], 0.0)
    deg = deg_ref[...] + 1.0
    dinv = lax.rsqrt(deg)
    hw = jnp.dot(h2, w_ref[...], preferred_element_type=jnp.float32)
    hv_ref[...] = dinv * hw
    dinv_ref[...] = jnp.broadcast_to(dinv, hw.shape)


def _tc_gcn_prep(h, parts, g, be, deg, w):
    return pl.pallas_call(
        _gcn_prep_body,
        grid=(N // _RB,),
        in_specs=[
            _nspec((_RB, D), lambda i: (i, 0)),
            _nspec((2, _RB, D), lambda i: (0, i, 0)),
            _nspec((1, D), lambda i: (0, 0)),
            _nspec((1, D), lambda i: (0, 0)),
            _nspec((_RB, 1), lambda i: (i, 0)),
            _nspec((D, D), lambda i: (0, 0)),
        ],
        out_specs=[
            _nspec((_RB, D), lambda i: (i, 0)),
            _nspec((_RB, D), lambda i: (i, 0)),
        ],
        out_shape=[
            jax.ShapeDtypeStruct((N, D), jnp.float32),
            jax.ShapeDtypeStruct((N, D), jnp.float32),
        ],
        interpret=_INTERPRET,
    )(h, parts, g.reshape(1, D), be.reshape(1, D), deg, w)


def _gcn_mid_body(p_ref, hv_ref, dinv_ref, b_ref, w_ref, hv4_ref):
    agg = p_ref[0] + p_ref[1] + hv_ref[...]
    h3 = jnp.maximum(dinv_ref[...] * agg + b_ref[...], 0.0)
    hv4_ref[...] = dinv_ref[...] * jnp.dot(
        h3, w_ref[...], preferred_element_type=jnp.float32)


def _tc_gcn_mid(parts, hv, dinv, b, w):
    return pl.pallas_call(
        _gcn_mid_body,
        grid=(N // _RB,),
        in_specs=[
            _nspec((2, _RB, D), lambda i: (0, i, 0)),
            _nspec((_RB, D), lambda i: (i, 0)),
            _nspec((_RB, D), lambda i: (i, 0)),
            _nspec((1, D), lambda i: (0, 0)),
            _nspec((D, D), lambda i: (0, 0)),
        ],
        out_specs=_nspec((_RB, D), lambda i: (i, 0)),
        out_shape=jax.ShapeDtypeStruct((N, D), jnp.float32),
        interpret=_INTERPRET,
    )(parts, hv, dinv, b.reshape(1, D), w)


def _final_body(p_ref, hv_ref, dinv_ref, b4_ref, goal_ref, d1wh_ref,
                d1wg_ref, d1b_ref, d2w_ref, d2b_ref, o_ref):
    agg = p_ref[0] + p_ref[1] + hv_ref[...]
    h4 = jnp.maximum(dinv_ref[...] * agg + b4_ref[...], 0.0)
    gterm = jnp.dot(goal_ref[...], d1wg_ref[...],
                    preferred_element_type=jnp.float32) + d1b_ref[...]
    hid = jnp.maximum(
        jnp.dot(h4, d1wh_ref[...], preferred_element_type=jnp.float32)
        + gterm, 0.0)
    o_ref[...] = jnp.dot(hid, d2w_ref[...],
                         preferred_element_type=jnp.float32) + d2b_ref[...]


def _tc_final(parts, hv, dinv, b4, goal, d1wh, d1wg, d1b, d2w, d2b):
    return pl.pallas_call(
        _final_body,
        grid=(N // _RB,),
        in_specs=[
            _nspec((2, _RB, D), lambda i: (0, i, 0)),
            _nspec((_RB, D), lambda i: (i, 0)),
            _nspec((_RB, D), lambda i: (i, 0)),
            _nspec((1, D), lambda i: (0, 0)),
            _nspec((1, D), lambda i: (0, 0)),
            _nspec((D, D), lambda i: (0, 0)),
            _nspec((D, D), lambda i: (0, 0)),
            _nspec((1, D), lambda i: (0, 0)),
            _nspec((D, 1), lambda i: (0, 0)),
            _nspec((1, 1), lambda i: (0, 0)),
        ],
        out_specs=_nspec((_RB, 1), lambda i: (i, 0)),
        out_shape=jax.ShapeDtypeStruct((N, 1), jnp.float32),
        interpret=_INTERPRET,
    )(parts, hv, dinv, b4.reshape(1, D), goal, d1wh, d1wg,
      d1b.reshape(1, D), d2w, d2b.reshape(1, 1))


_SC_MESH = plsc.VectorSubcoreMesh(core_axis_name="c", subcore_axis_name="s",
                                  num_cores=_NC, num_subcores=_NS)


def _rowcopy(sid, src_ref, dst_ref):
    pltpu.sync_copy(src_ref.at[pl.ds(sid * _RT, _RT)],
                    dst_ref.at[pl.ds(sid * _RT, _RT)])

    @pl.when(sid == _NS - 1)
    def _():
        pltpu.sync_copy(src_ref.at[pl.ds(_RT * _NS, _TAIL)],
                        dst_ref.at[pl.ds(_RT * _NS, _TAIL)])


def _scat_body(msg_hbm, dstx_hbm, zero_hbm, out_hbm, didx, buf, accum):
    cid = lax.axis_index("c")
    sid = lax.axis_index("s")
    base = (cid * _NS + sid) * _EPT
    pltpu.sync_copy(dstx_hbm.at[cid, sid], didx)
    _rowcopy(sid, zero_hbm, accum)
    plsc.subcore_barrier()

    def step(j, carry):
        pltpu.sync_copy(msg_hbm.at[pl.ds(base + j * _C, _C)], buf)
        pltpu.sync_copy(buf, accum.at[didx.at[j]], add=True)
        return carry

    lax.fori_loop(0, _NCH, step, 0)
    plsc.subcore_barrier()
    _rowcopy(sid, accum, out_hbm.at[cid])


_sc_scat = pl.kernel(
    _scat_body,
    out_type=jax.ShapeDtypeStruct((_NC, N, D), jnp.float32),
    mesh=_SC_MESH,
    scratch_types=[
        pltpu.VMEM((_NCH, _C), jnp.int32),
        pltpu.VMEM((_C, D), jnp.float32),
        pltpu.VMEM_SHARED((N, D), jnp.float32),
    ],
)


def _gcn_body(hv_hbm, srcx_hbm, dstx_hbm, zero_hbm, out_hbm,
              sidx, didx, buf, accum):
    cid = lax.axis_index("c")
    sid = lax.axis_index("s")
    pltpu.sync_copy(srcx_hbm.at[cid, sid], sidx)
    pltpu.sync_copy(dstx_hbm.at[cid, sid], didx)
    _rowcopy(sid, zero_hbm, accum)
    plsc.subcore_barrier()

    def step(j, carry):
        pltpu.sync_copy(hv_hbm.at[sidx.at[j]], buf)
        pltpu.sync_copy(buf, accum.at[didx.at[j]], add=True)
        return carry

    lax.fori_loop(0, _NCH, step, 0)
    plsc.subcore_barrier()
    _rowcopy(sid, accum, out_hbm.at[cid])


_sc_gcn = pl.kernel(
    _gcn_body,
    out_type=jax.ShapeDtypeStruct((_NC, N, D), jnp.float32),
    mesh=_SC_MESH,
    scratch_types=[
        pltpu.VMEM((_NCH, _C), jnp.int32),
        pltpu.VMEM((_NCH, _C), jnp.int32),
        pltpu.VMEM((_C, D), jnp.float32),
        pltpu.VMEM_SHARED((N, D), jnp.float32),
    ],
)


def _ga_body(td_hbm, ts_hbm, dstx_hbm, srcx_hbm, outd_hbm, outs_hbm,
             didx, sidx, bufa, bufb, sga, sgb, swa, swb):
    cid = lax.axis_index("c")
    sid = lax.axis_index("s")
    base = (cid * _NS + sid) * _EPT
    pltpu.sync_copy(dstx_hbm.at[cid, sid], didx)
    pltpu.sync_copy(srcx_hbm.at[cid, sid], sidx)

    def step(j, carry):
        rows = pl.ds(base + j * _C, _C)
        ca = pltpu.async_copy(td_hbm.at[didx.at[j]], bufa, sga)
        cb = pltpu.async_copy(ts_hbm.at[sidx.at[j]], bufb, sgb)
        ca.wait()
        cb.wait()
        wa = pltpu.async_copy(bufa, outd_hbm.at[rows], swa)
        wb = pltpu.async_copy(bufb, outs_hbm.at[rows], swb)
        wa.wait()
        wb.wait()
        return carry

    lax.fori_loop(0, _NCH, step, 0)


_sc_gather_add = pl.kernel(
    _ga_body,
    out_type=[jax.ShapeDtypeStruct((E, D), jnp.uint32),
              jax.ShapeDtypeStruct((E, D), jnp.uint32)],
    mesh=_SC_MESH,
    scratch_types=[
        pltpu.VMEM((_NCH, _C), jnp.int32),
        pltpu.VMEM((_NCH, _C), jnp.int32),
        pltpu.VMEM((_C, D), jnp.uint32),
        pltpu.VMEM((_C, D), jnp.uint32),
        pltpu.SemaphoreType.DMA,
        pltpu.SemaphoreType.DMA,
        pltpu.SemaphoreType.DMA,
        pltpu.SemaphoreType.DMA,
    ],
)


def _gather_add(td, ts, dst, src):
    return td[dst] + ts[src]


def kernel(x, edge_index, edge_attr, batch, goal_feat, f1W, f1b, s1W, s1b,
           g1, be1, f2W, f2b, s2W, s2b, g2, be2, W3, b3, W4, b4,
           D1W, D1b, D2W, D2b):
    src = edge_index[0]
    dst = edge_index[1]
    srcx = src.reshape(_NC, _NS, _NCH, _C)
    dstx = dst.reshape(_NC, _NS, _NCH, _C)
    zero_nd = jnp.zeros((N, D), jnp.float32)
    deg = jax.ops.segment_sum(jnp.ones((E,), jnp.float32), dst,
                              num_segments=N).reshape(N, 1)

    wd1 = jnp.concatenate([f1W[:D], s1W[:D]], axis=1)
    ws1 = jnp.concatenate([f1W[D:2 * D], s1W[D:2 * D]], axis=1)
    we1 = jnp.concatenate([f1W[2 * D:], s1W[2 * D:]], axis=1)
    bfs1 = jnp.concatenate([f1b, s1b])
    wd2 = jnp.concatenate([f2W[:D], s2W[:D]], axis=1)
    ws2 = jnp.concatenate([f2W[D:2 * D], s2W[D:2 * D]], axis=1)
    we2 = jnp.concatenate([f2W[2 * D:], s2W[2 * D:]], axis=1)
    bfs2 = jnp.concatenate([f2b, s2b])

    td1, ts1 = _tc_pre(x, wd1, ws1)
    fd1, fs1 = _sc_gather_add(td1, ts1, dstx, srcx)
    msg1 = _tc_act(fd1, fs1, edge_attr, we1, bfs1)
    parts1 = _sc_scat(msg1, dstx, zero_nd)

    h1, td2, ts2 = _tc_mid(x, parts1, g1, be1, wd2, ws2)
    fd2, fs2 = _sc_gather_add(td2, ts2, dstx, srcx)
    msg2 = _tc_act(fd2, fs2, edge_attr, we2, bfs2)
    parts2 = _sc_scat(msg2, dstx, zero_nd)

    hv3, dinv = _tc_gcn_prep(h1, parts2, g2, be2, deg, W3)
    parts3 = _sc_gcn(hv3, srcx, dstx, zero_nd)

    hv4 = _tc_gcn_mid(parts3, hv3, dinv, b3, W4)
    parts4 = _sc_gcn(hv4, srcx, dstx, zero_nd)

    return _tc_final(parts4, hv4, dinv, b4, goal_feat, D1W[:D], D1W[D:],
                     D1b, D2W, D2b)


# speedup vs baseline: 6.6979x; 1.0260x over previous
import functools
import math

import jax
import jax.numpy as jnp
from jax import lax
from jax.experimental import pallas as pl
from jax.experimental.pallas import tpu as pltpu
from jax.experimental.pallas import tpu_sc as plsc

N = 10000
E = 320000
D = 128
DE = 16

_NC = 2
_NS = 16
_NW = _NC * _NS
_EPT = E // _NW
_C = 80
_NCH = _EPT // _C
_RT = 624
_TAIL = N - _RT * _NS
_C2 = 40
_NCH2 = _EPT // _C2

_BN = 1.0 / math.sqrt(1.0 + 1e-5)
_RB = 2000
_EB = 4000

_INTERPRET = False


def _nspec(shape, imap):
    return pl.BlockSpec(shape, imap)


def _pack2(t):
    f = lax.bitcast_convert_type(t[:, :D], jnp.uint32)
    s = lax.bitcast_convert_type(t[:, D:], jnp.uint32)
    f = (f + jnp.uint32(0x7FFF) + ((f >> 16) & jnp.uint32(1))) >> 16
    s = (s + jnp.uint32(0x7FFF) + ((s >> 16) & jnp.uint32(1))) & jnp.uint32(
        0xFFFF0000)
    return f | s


def _unpack_f(w):
    return lax.bitcast_convert_type(w << 16, jnp.float32)


def _unpack_s(w):
    return lax.bitcast_convert_type(w & jnp.uint32(0xFFFF0000), jnp.float32)


def _pre_body(x_ref, wd_ref, ws_ref, td_ref, ts_ref):
    xb = x_ref[...]
    td_ref[...] = _pack2(
        jnp.dot(xb, wd_ref[...], preferred_element_type=jnp.float32))
    ts_ref[...] = _pack2(
        jnp.dot(xb, ws_ref[...], preferred_element_type=jnp.float32))


def _tc_pre(x, wd, ws):
    return pl.pallas_call(
        _pre_body,
        grid=(N // _RB,),
        in_specs=[
            _nspec((_RB, D), lambda i: (i, 0)),
            _nspec((D, 2 * D), lambda i: (0, 0)),
            _nspec((D, 2 * D), lambda i: (0, 0)),
        ],
        out_specs=[
            _nspec((_RB, D), lambda i: (i, 0)),
            _nspec((_RB, D), lambda i: (i, 0)),
        ],
        out_shape=[
            jax.ShapeDtypeStruct((N, D), jnp.uint32),
            jax.ShapeDtypeStruct((N, D), jnp.uint32),
        ],
        interpret=_INTERPRET,
    )(x, wd, ws)


def _mid_body(x_ref, p_ref, g_ref, be_ref, wd_ref, ws_ref,
              h_ref, td_ref, ts_ref):
    agg = (p_ref[0] + p_ref[1]) * _BN
    h = jnp.maximum(x_ref[...] + g_ref[...] * agg + be_ref[...], 0.0)
    h_ref[...] = h
    td_ref[...] = _pack2(
        jnp.dot(h, wd_ref[...], preferred_element_type=jnp.float32))
    ts_ref[...] = _pack2(
        jnp.dot(h, ws_ref[...], preferred_element_type=jnp.float32))


def _tc_mid(x, parts, g, be, wd, ws):
    return pl.pallas_call(
        _mid_body,
        grid=(N // _RB,),
        in_specs=[
            _nspec((_RB, D), lambda i: (i, 0)),
            _nspec((2, _RB, D), lambda i: (0, i, 0)),
            _nspec((1, D), lambda i: (0, 0)),
            _nspec((1, D), lambda i: (0, 0)),
            _nspec((D, 2 * D), lambda i: (0, 0)),
            _nspec((D, 2 * D), lambda i: (0, 0)),
        ],
        out_specs=[
            _nspec((_RB, D), lambda i: (i, 0)),
            _nspec((_RB, D), lambda i: (i, 0)),
            _nspec((_RB, D), lambda i: (i, 0)),
        ],
        out_shape=[
            jax.ShapeDtypeStruct((N, D), jnp.float32),
            jax.ShapeDtypeStruct((N, D), jnp.uint32),
            jax.ShapeDtypeStruct((N, D), jnp.uint32),
        ],
        interpret=_INTERPRET,
    )(x, parts, g.reshape(1, D), be.reshape(1, D), wd, ws)


def _act_body(fd_ref, fs_ref, ea_ref, we_ref, bfs_ref, o_ref):
    wd_ = fd_ref[...]
    ws_ = fs_ref[...]
    g = jnp.dot(ea_ref[...], we_ref[...],
                preferred_element_type=jnp.float32) + bfs_ref[...]
    f = _unpack_f(wd_) + _unpack_f(ws_) + g[:, :D]
    s = _unpack_s(wd_) + _unpack_s(ws_) + g[:, D:]
    o_ref[...] = jax.nn.sigmoid(f) * jax.nn.softplus(s)


def _tc_act(fd, fs, ea, we, bfs):
    return pl.pallas_call(
        _act_body,
        grid=(E // _EB,),
        in_specs=[
            _nspec((_EB, D), lambda i: (i, 0)),
            _nspec((_EB, D), lambda i: (i, 0)),
            _nspec((_EB, DE), lambda i: (i, 0)),
            _nspec((DE, 2 * D), lambda i: (0, 0)),
            _nspec((1, 2 * D), lambda i: (0, 0)),
        ],
        out_specs=_nspec((_EB, D), lambda i: (i, 0)),
        out_shape=jax.ShapeDtypeStruct((E, D), jnp.float32),
        interpret=_INTERPRET,
    )(fd, fs, ea, we, bfs.reshape(1, 2 * D))


def _gcn_prep_body(h_ref, p_ref, g_ref, be_ref, deg_ref, w_ref,
                   hv_ref, dinv_ref):
    agg = (p_ref[0] + p_ref[1]) * _BN
    h2 = jnp.maximum(h_ref[...] + g_ref[...] * agg + be_ref[...], 0.0)
    deg = deg_ref[...] + 1.0
    dinv = lax.rsqrt(deg)
    hw = jnp.dot(h2, w_ref[...], preferred_element_type=jnp.float32)
    hv_ref[...] = dinv * hw
    dinv_ref[...] = jnp.broadcast_to(dinv, hw.shape)


def _tc_gcn_prep(h, parts, g, be, deg, w):
    return pl.pallas_call(
        _gcn_prep_body,
        grid=(N // _RB,),
        in_specs=[
            _nspec((_RB, D), lambda i: (i, 0)),
            _nspec((2, _RB, D), lambda i: (0, i, 0)),
            _nspec((1, D), lambda i: (0, 0)),
            _nspec((1, D), lambda i: (0, 0)),
            _nspec((_RB, 1), lambda i: (i, 0)),
            _nspec((D, D), lambda i: (0, 0)),
        ],
        out_specs=[
            _nspec((_RB, D), lambda i: (i, 0)),
            _nspec((_RB, D), lambda i: (i, 0)),
        ],
        out_shape=[
            jax.ShapeDtypeStruct((N, D), jnp.float32),
            jax.ShapeDtypeStruct((N, D), jnp.float32),
        ],
        interpret=_INTERPRET,
    )(h, parts, g.reshape(1, D), be.reshape(1, D), deg, w)


def _gcn_mid_body(p_ref, hv_ref, dinv_ref, b_ref, w_ref, hv4_ref):
    agg = p_ref[0] + p_ref[1] + hv_ref[...]
    h3 = jnp.maximum(dinv_ref[...] * agg + b_ref[...], 0.0)
    hv4_ref[...] = dinv_ref[...] * jnp.dot(
        h3, w_ref[...], preferred_element_type=jnp.float32)


def _tc_gcn_mid(parts, hv, dinv, b, w):
    return pl.pallas_call(
        _gcn_mid_body,
        grid=(N // _RB,),
        in_specs=[
            _nspec((2, _RB, D), lambda i: (0, i, 0)),
            _nspec((_RB, D), lambda i: (i, 0)),
            _nspec((_RB, D), lambda i: (i, 0)),
            _nspec((1, D), lambda i: (0, 0)),
            _nspec((D, D), lambda i: (0, 0)),
        ],
        out_specs=_nspec((_RB, D), lambda i: (i, 0)),
        out_shape=jax.ShapeDtypeStruct((N, D), jnp.float32),
        interpret=_INTERPRET,
    )(parts, hv, dinv, b.reshape(1, D), w)


def _final_body(p_ref, hv_ref, dinv_ref, b4_ref, goal_ref, d1wh_ref,
                d1wg_ref, d1b_ref, d2w_ref, d2b_ref, o_ref):
    agg = p_ref[0] + p_ref[1] + hv_ref[...]
    h4 = jnp.maximum(dinv_ref[...] * agg + b4_ref[...], 0.0)
    gterm = jnp.dot(goal_ref[...], d1wg_ref[...],
                    preferred_element_type=jnp.float32) + d1b_ref[...]
    hid = jnp.maximum(
        jnp.dot(h4, d1wh_ref[...], preferred_element_type=jnp.float32)
        + gterm, 0.0)
    o_ref[...] = jnp.dot(hid, d2w_ref[...],
                         preferred_element_type=jnp.float32) + d2b_ref[...]


def _tc_final(parts, hv, dinv, b4, goal, d1wh, d1wg, d1b, d2w, d2b):
    return pl.pallas_call(
        _final_body,
        grid=(N // _RB,),
        in_specs=[
            _nspec((2, _RB, D), lambda i: (0, i, 0)),
            _nspec((_RB, D), lambda i: (i, 0)),
            _nspec((_RB, D), lambda i: (i, 0)),
            _nspec((1, D), lambda i: (0, 0)),
            _nspec((1, D), lambda i: (0, 0)),
            _nspec((D, D), lambda i: (0, 0)),
            _nspec((D, D), lambda i: (0, 0)),
            _nspec((1, D), lambda i: (0, 0)),
            _nspec((D, 1), lambda i: (0, 0)),
            _nspec((1, 1), lambda i: (0, 0)),
        ],
        out_specs=_nspec((_RB, 1), lambda i: (i, 0)),
        out_shape=jax.ShapeDtypeStruct((N, 1), jnp.float32),
        interpret=_INTERPRET,
    )(parts, hv, dinv, b4.reshape(1, D), goal, d1wh, d1wg,
      d1b.reshape(1, D), d2w, d2b.reshape(1, 1))


_SC_MESH = plsc.VectorSubcoreMesh(core_axis_name="c", subcore_axis_name="s",
                                  num_cores=_NC, num_subcores=_NS)


def _rowcopy(sid, src_ref, dst_ref):
    pltpu.sync_copy(src_ref.at[pl.ds(sid * _RT, _RT)],
                    dst_ref.at[pl.ds(sid * _RT, _RT)])

    @pl.when(sid == _NS - 1)
    def _():
        pltpu.sync_copy(src_ref.at[pl.ds(_RT * _NS, _TAIL)],
                        dst_ref.at[pl.ds(_RT * _NS, _TAIL)])


def _scat_body(msg_hbm, dstx_hbm, zero_hbm, out_hbm, didx, buf0, accum):
    cid = lax.axis_index("c")
    sid = lax.axis_index("s")
    base = (cid * _NS + sid) * _EPT
    pltpu.sync_copy(dstx_hbm.at[cid, sid], didx)
    _rowcopy(sid, zero_hbm, accum)
    plsc.subcore_barrier()

    def step(j, carry):
        pltpu.sync_copy(msg_hbm.at[pl.ds(base + j * _C, _C)], buf0)
        pltpu.sync_copy(buf0, accum.at[didx.at[j]], add=True)
        return carry

    lax.fori_loop(0, _NCH, step, 0)
    plsc.subcore_barrier()
    _rowcopy(sid, accum, out_hbm.at[cid])


_sc_scat = pl.kernel(
    _scat_body,
    out_type=jax.ShapeDtypeStruct((_NC, N, D), jnp.float32),
    mesh=_SC_MESH,
    scratch_types=[
        pltpu.VMEM((_NCH, _C), jnp.int32),
        pltpu.VMEM((_C, D), jnp.float32),
        pltpu.VMEM_SHARED((N, D), jnp.float32),
    ],
)


def _gcn_body(hv_hbm, srcx_hbm, dstx_hbm, zero_hbm, out_hbm,
              sidx, didx, buf0, accum):
    cid = lax.axis_index("c")
    sid = lax.axis_index("s")
    pltpu.sync_copy(srcx_hbm.at[cid, sid], sidx)
    pltpu.sync_copy(dstx_hbm.at[cid, sid], didx)
    _rowcopy(sid, zero_hbm, accum)
    plsc.subcore_barrier()

    def step(j, carry):
        pltpu.sync_copy(hv_hbm.at[sidx.at[j]], buf0)
        pltpu.sync_copy(buf0, accum.at[didx.at[j]], add=True)
        return carry

    lax.fori_loop(0, _NCH, step, 0)
    plsc.subcore_barrier()
    _rowcopy(sid, accum, out_hbm.at[cid])


_sc_gcn = pl.kernel(
    _gcn_body,
    out_type=jax.ShapeDtypeStruct((_NC, N, D), jnp.float32),
    mesh=_SC_MESH,
    scratch_types=[
        pltpu.VMEM((_NCH, _C), jnp.int32),
        pltpu.VMEM((_NCH, _C), jnp.int32),
        pltpu.VMEM((_C, D), jnp.float32),
        pltpu.VMEM_SHARED((N, D), jnp.float32),
    ],
)


def _ga_body(td_hbm, ts_hbm, dstx_hbm, srcx_hbm, outd_hbm, outs_hbm,
             didx, sidx, bufa0, bufa1, bufb0, bufb1,
             ga0, ga1, gb0, gb1, wa0, wa1, wb0, wb1):
    cid = lax.axis_index("c")
    sid = lax.axis_index("s")
    base = (cid * _NS + sid) * _EPT
    pltpu.sync_copy(dstx_hbm.at[cid, sid], didx)
    pltpu.sync_copy(srcx_hbm.at[cid, sid], sidx)

    bas = (bufa0, bufa1)
    bbs = (bufb0, bufb1)
    gas = (ga0, ga1)
    gbs = (gb0, gb1)
    was = (wa0, wa1)
    wbs = (wb0, wb1)
    pltpu.async_copy(td_hbm.at[didx.at[0]], bufa0, ga0)
    pltpu.async_copy(ts_hbm.at[sidx.at[0]], bufb0, gb0)

    def steppair(jj, carry):
        for b in range(2):
            j = jj * 2 + b
            nb = 1 - b
            rows = pl.ds(base + j * _C, _C)
            pltpu.make_async_copy(td_hbm.at[didx.at[j]], bas[b],
                                  gas[b]).wait()
            pltpu.make_async_copy(ts_hbm.at[sidx.at[j]], bbs[b],
                                  gbs[b]).wait()

            @pl.when(j > 0)
            def _():
                pltpu.make_async_copy(bas[nb], outd_hbm.at[rows],
                                      was[nb]).wait()
                pltpu.make_async_copy(bbs[nb], outs_hbm.at[rows],
                                      wbs[nb]).wait()

            @pl.when(j + 1 < _NCH)
            def _():
                pltpu.async_copy(td_hbm.at[didx.at[j + 1]], bas[nb], gas[nb])
                pltpu.async_copy(ts_hbm.at[sidx.at[j + 1]], bbs[nb], gbs[nb])

            pltpu.async_copy(bas[b], outd_hbm.at[rows], was[b])
            pltpu.async_copy(bbs[b], outs_hbm.at[rows], wbs[b])
        return carry

    lax.fori_loop(0, _NCH // 2, steppair, 0)
    last = pl.ds(base + (_NCH - 1) * _C, _C)
    pltpu.make_async_copy(td_hbm.at[didx.at[0]], bufa0, ga0).wait()
    pltpu.make_async_copy(ts_hbm.at[sidx.at[0]], bufb0, gb0).wait()
    pltpu.make_async_copy(bufa1, outd_hbm.at[last], wa1).wait()
    pltpu.make_async_copy(bufb1, outs_hbm.at[last], wb1).wait()
    pltpu.sync_copy(bufa0, outd_hbm.at[last])
    pltpu.sync_copy(bufb0, outs_hbm.at[last])


_sc_gather_add = pl.kernel(
    _ga_body,
    out_type=[jax.ShapeDtypeStruct((E, D), jnp.uint32),
              jax.ShapeDtypeStruct((E, D), jnp.uint32)],
    mesh=_SC_MESH,
    scratch_types=[
        pltpu.VMEM((_NCH, _C), jnp.int32),
        pltpu.VMEM((_NCH, _C), jnp.int32),
        pltpu.VMEM((_C, D), jnp.uint32),
        pltpu.VMEM((_C, D), jnp.uint32),
        pltpu.VMEM((_C, D), jnp.uint32),
        pltpu.VMEM((_C, D), jnp.uint32),
        pltpu.SemaphoreType.DMA,
        pltpu.SemaphoreType.DMA,
        pltpu.SemaphoreType.DMA,
        pltpu.SemaphoreType.DMA,
        pltpu.SemaphoreType.DMA,
        pltpu.SemaphoreType.DMA,
        pltpu.SemaphoreType.DMA,
        pltpu.SemaphoreType.DMA,
    ],
)


def _gather_add(td, ts, dst, src):
    return td[dst] + ts[src]


def kernel(x, edge_index, edge_attr, batch, goal_feat, f1W, f1b, s1W, s1b,
           g1, be1, f2W, f2b, s2W, s2b, g2, be2, W3, b3, W4, b4,
           D1W, D1b, D2W, D2b):
    src = edge_index[0]
    dst = edge_index[1]
    srcx = src.reshape(_NC, _NS, _NCH, _C)
    dstx = dst.reshape(_NC, _NS, _NCH, _C)
    srcx2 = src.reshape(_NC, _NS, _NCH2, _C2)
    dstx2 = dst.reshape(_NC, _NS, _NCH2, _C2)
    zero_nd = jnp.zeros((N, D), jnp.float32)
    deg = jax.ops.segment_sum(jnp.ones((E,), jnp.float32), dst,
                              num_segments=N).reshape(N, 1)

    wd1 = jnp.concatenate([f1W[:D], s1W[:D]], axis=1)
    ws1 = jnp.concatenate([f1W[D:2 * D], s1W[D:2 * D]], axis=1)
    we1 = jnp.concatenate([f1W[2 * D:], s1W[2 * D:]], axis=1)
    bfs1 = jnp.concatenate([f1b, s1b])
    wd2 = jnp.concatenate([f2W[:D], s2W[:D]], axis=1)
    ws2 = jnp.concatenate([f2W[D:2 * D], s2W[D:2 * D]], axis=1)
    we2 = jnp.concatenate([f2W[2 * D:], s2W[2 * D:]], axis=1)
    bfs2 = jnp.concatenate([f2b, s2b])

    td1, ts1 = _tc_pre(x, wd1, ws1)
    fd1, fs1 = _sc_gather_add(td1, ts1, dstx, srcx)
    msg1 = _tc_act(fd1, fs1, edge_attr, we1, bfs1)
    parts1 = _sc_scat(msg1, dstx, zero_nd)

    h1, td2, ts2 = _tc_mid(x, parts1, g1, be1, wd2, ws2)
    fd2, fs2 = _sc_gather_add(td2, ts2, dstx, srcx)
    msg2 = _tc_act(fd2, fs2, edge_attr, we2, bfs2)
    parts2 = _sc_scat(msg2, dstx, zero_nd)

    hv3, dinv = _tc_gcn_prep(h1, parts2, g2, be2, deg, W3)
    parts3 = _sc_gcn(hv3, srcx, dstx, zero_nd)

    hv4 = _tc_gcn_mid(parts3, hv3, dinv, b3, W4)
    parts4 = _sc_gcn(hv4, srcx, dstx, zero_nd)

    return _tc_final(parts4, hv4, dinv, b4, goal_feat, D1W[:D], D1W[D:],
                     D1b, D2W, D2b)
